# baseline (speedup 1.0000x reference)
import jax
import jax.numpy as jnp
from jax import lax
from jax.experimental import pallas as pl
from jax.experimental.pallas import tpu as pltpu
from jax.experimental.pallas import tpu_sc as plsc

N = 10000
E = 320000
D = 128
NK = 2
H = 128
OUT = 64

NC = 2
NS = 16
NW = NC * NS
EPT = E // NW
NPAD = 10240
NRG = 64
RR = NPAD // NRG
NQ = 8
QR = NPAD // NQ
SB = 2048
REPT = EPT + SB + 16
ROUNDS = NRG // NW

_mesh = plsc.VectorSubcoreMesh(core_axis_name="c", subcore_axis_name="s",
                               num_cores=NC, num_subcores=NS)
_sc_params = pltpu.CompilerParams(needs_layout_passes=False)


def _route_body(pki, k0i, k1i, rt_pk, rt_k0, rt_k1, rt_base,
                m_pk, m_k0, m_k1, m2_pk, m2_k0, m2_k1, bb):
    c = lax.axis_index("c")
    s = lax.axis_index("s")
    wid = s * NC + c
    ebase = wid * EPT
    zvec_i = jnp.zeros((16,), jnp.int32)

    pltpu.sync_copy(pki.at[pl.ds(pl.multiple_of(ebase, 8), EPT)],
                    m_pk.at[pl.ds(0, EPT)])
    pltpu.sync_copy(k0i.at[pl.ds(pl.multiple_of(ebase, 8), EPT)],
                    m_k0.at[pl.ds(0, EPT)])
    pltpu.sync_copy(k1i.at[pl.ds(pl.multiple_of(ebase, 8), EPT)],
                    m_k1.at[pl.ds(0, EPT)])

    def cnt1(v, cr):
        d = m_pk[pl.ds(v * 16, 16)] >> 16
        return tuple(
            cr[q] + plsc.all_reduce_population_count(
                (d >= q * QR) & (d < (q + 1) * QR))
            for q in range(NQ))

    qcnt = lax.fori_loop(0, EPT // 16, cnt1, (zvec_i,) * NQ)
    qbase = [zvec_i]
    for q in range(NQ - 1):
        qbase.append(qbase[q] + qcnt[q])

    def fill1(v, w):
        sl = pl.ds(v * 16, 16)
        pk = m_pk[sl]
        kv0 = m_k0[sl]
        kv1 = m_k1[sl]
        d = pk >> 16
        ws = list(w)
        for q in range(NQ):
            m = (d >= q * QR) & (d < (q + 1) * QR)
            mi = m.astype(jnp.int32)
            posv = ws[q] + lax.cumsum(mi) - mi
            plsc.store_scatter(m2_pk, [posv], pk, mask=m)
            plsc.store_scatter(m2_k0, [posv], kv0, mask=m)
            plsc.store_scatter(m2_k1, [posv], kv1, mask=m)
            ws[q] = ws[q] + plsc.all_reduce_population_count(m)
        return tuple(ws)

    lax.fori_loop(0, EPT // 16, fill1, tuple(qbase))

    iota16 = lax.iota(jnp.int32, 16)
    gbase_splat = []
    for q in range(NQ):
        qb = qbase[q][0]
        qe = qbase[q + 1][0] if q + 1 < NQ else jnp.int32(EPT)
        qa = pl.multiple_of(qb & jnp.int32(-8), 8)
        nv = (qe - qa + 15) // 16
        qbv = qbase[q]

        def cnt2(v, cr, qa=qa, qb=qb, qe=qe, q=q):
            voff = pl.multiple_of(qa + v * 16, 8)
            d = m2_pk[pl.ds(voff, 16)] >> 16
            posn = voff + iota16
            ok = (posn >= qb) & (posn < qe)
            return tuple(
                cr[k] + plsc.all_reduce_population_count(
                    ok & (d >= q * QR + k * RR) & (d < q * QR + (k + 1) * RR))
                for k in range(NQ))

        scnt = lax.fori_loop(0, nv, cnt2, (zvec_i,) * NQ)
        sbase = [qbv]
        for k in range(NQ - 1):
            sbase.append(sbase[k] + scnt[k])
        gbase_splat.extend(sbase)

        def fill2(v, w, qa=qa, qb=qb, qe=qe, q=q):
            voff = pl.multiple_of(qa + v * 16, 8)
            sl = pl.ds(voff, 16)
            pk = m2_pk[sl]
            kv0 = m2_k0[sl]
            kv1 = m2_k1[sl]
            d = pk >> 16
            posn = voff + iota16
            ok = (posn >= qb) & (posn < qe)
            ws = list(w)
            for k in range(NQ):
                m = ok & (d >= q * QR + k * RR) & (d < q * QR + (k + 1) * RR)
                mi = m.astype(jnp.int32)
                posv = ws[k] + lax.cumsum(mi) - mi
                plsc.store_scatter(m_pk, [posv], pk, mask=m)
                plsc.store_scatter(m_k0, [posv], kv0, mask=m)
                plsc.store_scatter(m_k1, [posv], kv1, mask=m)
                ws[k] = ws[k] + plsc.all_reduce_population_count(m)
            return tuple(ws)

        lax.fori_loop(0, nv, fill2, tuple(sbase))

    zvec_f = jnp.zeros((16,), jnp.float32)

    def ztail(i, _):
        t = pl.multiple_of(EPT + i * 16, 8)
        m_pk[pl.ds(t, 16)] = zvec_i
        m_k0[pl.ds(t, 16)] = zvec_f
        m_k1[pl.ds(t, 16)] = zvec_f
        return 0

    lax.fori_loop(0, (REPT - EPT) // 16, ztail, 0)

    lane0 = iota16 == 0
    for g in range(NRG):
        plsc.store_scatter(bb, [zvec_i + g], gbase_splat[g], mask=lane0)
    plsc.store_scatter(bb, [zvec_i + NRG], zvec_i + EPT, mask=lane0)
    pltpu.sync_copy(bb, rt_base.at[pl.ds(pl.multiple_of(wid * 80, 8), 80)])
    rbase = pl.multiple_of(wid * REPT, 8)
    pltpu.sync_copy(m_pk, rt_pk.at[pl.ds(rbase, REPT)])
    pltpu.sync_copy(m_k0, rt_k0.at[pl.ds(rbase, REPT)])
    pltpu.sync_copy(m_k1, rt_k1.at[pl.ds(rbase, REPT)])


_route = pl.kernel(
    _route_body,
    out_type=(jax.ShapeDtypeStruct((NW * REPT,), jnp.int32),
              jax.ShapeDtypeStruct((NW * REPT,), jnp.float32),
              jax.ShapeDtypeStruct((NW * REPT,), jnp.float32),
              jax.ShapeDtypeStruct((NW * 80,), jnp.int32)),
    mesh=_mesh,
    compiler_params=_sc_params,
    scratch_types=[
        pltpu.VMEM((REPT,), jnp.int32),
        pltpu.VMEM((REPT,), jnp.float32),
        pltpu.VMEM((REPT,), jnp.float32),
        pltpu.VMEM((EPT + 48,), jnp.int32),
        pltpu.VMEM((EPT + 48,), jnp.float32),
        pltpu.VMEM((EPT + 48,), jnp.float32),
        pltpu.VMEM((80,), jnp.int32),
    ],
)


def _make_acc(Din, C):
    Dout = NK * Din
    assert SB % C == 0 and C % 16 == 0

    def body(xin, rt_pk, rt_k0, rt_k1, rt_base, xout,
             tbv, s_pk, s_k0, s_k1, gidx, rows, acc):
        c = lax.axis_index("c")
        s = lax.axis_index("s")
        wid = s * NC + c
        zvec_f = jnp.zeros((16,), jnp.float32)
        iota16 = lax.iota(jnp.int32, 16)

        pltpu.sync_copy(rt_base, tbv)
        uof = jnp.zeros((16,), jnp.int32)

        for r in range(ROUNDS):
            g = wid + NW * r
            range_lo = g * RR

            def zacc(i, _):
                for qq in range(Dout // 16):
                    acc[i, pl.ds(qq * 16, 16)] = zvec_f
                return 0

            lax.fori_loop(0, RR, zacc, 0)

            def u_body(u, _):
                b0 = plsc.load_gather(tbv, [uof + u * 80 + g])[0]
                b1 = plsc.load_gather(tbv, [uof + u * 80 + g + 1])[0]
                astart = pl.multiple_of(b0 & jnp.int32(-8), 8)
                nblk = (b1 - astart + SB - 1) // SB

                def blk_body(bi, _2):
                    boff = pl.multiple_of(astart + bi * SB, 8)
                    fo = pl.multiple_of(u * REPT + boff, 8)
                    pltpu.sync_copy(rt_pk.at[pl.ds(fo, SB)], s_pk)
                    pltpu.sync_copy(rt_k0.at[pl.ds(fo, SB)], s_k0)
                    pltpu.sync_copy(rt_k1.at[pl.ds(fo, SB)], s_k1)

                    def unpk(v, _3):
                        sl = pl.ds(v * 16, 16)
                        gidx[sl] = s_pk[sl] & jnp.int32(0xFFFF)
                        return 0

                    lax.fori_loop(0, SB // 16, unpk, 0)

                    def ch_body(cc, _3):
                        coff = pl.multiple_of(cc * C, 8)
                        pltpu.sync_copy(xin.at[gidx.at[pl.ds(coff, C)]], rows)

                        def vec_body(v2, _4):
                            voff = pl.multiple_of(coff + v2 * 16, 8)
                            sl = pl.ds(voff, 16)
                            posn = boff + voff + iota16
                            ok = (posn >= b0) & (posn < b1)
                            kv0 = jnp.where(ok, s_k0[sl], 0.0)
                            kv1 = jnp.where(ok, s_k1[sl], 0.0)
                            dv = s_pk[sl] >> 16
                            liv = jnp.clip(dv - range_lo, 0, RR - 1)
                            for e in range(16):
                                er = v2 * 16 + e
                                li = liv[e]
                                k0s = kv0[e]
                                k1s = kv1[e]
                                for qd in range(Din // 16):
                                    rv = rows[er, pl.ds(qd * 16, 16)]
                                    plsc.addupdate(
                                        acc.at[li, pl.ds(qd * 16, 16)],
                                        rv * k0s)
                                    plsc.addupdate(
                                        acc.at[li, pl.ds(Din + qd * 16, 16)],
                                        rv * k1s)
                            return 0

                        lax.fori_loop(0, C // 16, vec_body, 0)
                        return 0

                    lax.fori_loop(0, SB // C, ch_body, 0)
                    return 0

                lax.fori_loop(0, nblk, blk_body, 0)
                return 0

            lax.fori_loop(0, NW, u_body, 0)
            pltpu.sync_copy(
                acc, xout.at[pl.ds(pl.multiple_of(range_lo, 8), RR)])

    return pl.kernel(
        body,
        out_type=jax.ShapeDtypeStruct((NPAD, Dout), jnp.float32),
        mesh=_mesh,
        compiler_params=_sc_params,
        scratch_types=[
            pltpu.VMEM((NW * 80,), jnp.int32),
            pltpu.VMEM((SB,), jnp.int32),
            pltpu.VMEM((SB,), jnp.float32),
            pltpu.VMEM((SB,), jnp.float32),
            pltpu.VMEM((SB,), jnp.int32),
            pltpu.VMEM((C, Din), jnp.float32),
            pltpu.VMEM((RR, Dout), jnp.float32),
        ],
    )


_acc1 = _make_acc(Din=D, C=64)
_acc2 = _make_acc(Din=NK * D, C=64)

BLK = 1000


def _mlp_body(x_ref, x1_ref, x2_ref, w1_ref, b1_ref, w2_ref, b2_ref,
              enc_ref, out_ref):
    x = x_ref[...]
    nrm = jnp.sqrt(jnp.sum(x * x, axis=1, keepdims=True))
    out0 = x / jnp.maximum(nrm, 1e-12)
    x1 = x1_ref[...]
    x2 = x2_ref[...]
    h = (jnp.dot(out0, w1_ref[0:D, :], preferred_element_type=jnp.float32)
         + jnp.dot(x1, w1_ref[D:D + NK * D, :], preferred_element_type=jnp.float32)
         + jnp.dot(x2, w1_ref[D + NK * D:, :], preferred_element_type=jnp.float32)
         + b1_ref[...])
    h = jnp.maximum(h, 0.0)
    enc_ref[...] = jnp.dot(h, w2_ref[...], preferred_element_type=jnp.float32) + b2_ref[...]
    out_ref[:, 0:D] = out0
    out_ref[:, D:D + NK * D] = x1
    out_ref[:, D + NK * D:] = x2


def _mlp(x, x1, x2, W1, b1, W2, b2):
    total = D + NK * D + NK * NK * D
    grid = N // BLK
    return pl.pallas_call(
        _mlp_body,
        grid=(grid,),
        in_specs=[
            pl.BlockSpec((BLK, D), lambda i: (i, 0)),
            pl.BlockSpec((BLK, NK * D), lambda i: (i, 0)),
            pl.BlockSpec((BLK, NK * NK * D), lambda i: (i, 0)),
            pl.BlockSpec((total, H), lambda i: (0, 0)),
            pl.BlockSpec((1, H), lambda i: (0, 0)),
            pl.BlockSpec((H, OUT), lambda i: (0, 0)),
            pl.BlockSpec((1, OUT), lambda i: (0, 0)),
        ],
        out_specs=[
            pl.BlockSpec((BLK, OUT), lambda i: (i, 0)),
            pl.BlockSpec((BLK, total), lambda i: (i, 0)),
        ],
        out_shape=[
            jax.ShapeDtypeStruct((N, OUT), jnp.float32),
            jax.ShapeDtypeStruct((N, total), jnp.float32),
        ],
    )(x, x1, x2, W1, b1.reshape(1, H), W2, b2.reshape(1, OUT))


@jax.jit
def kernel(x, edge_index, K_vals, W1, b1, W2, b2):
    src = edge_index[0]
    dst = edge_index[1]
    pk = src | (dst << 16)
    rt_pk, rt_k0, rt_k1, rt_base = _route(pk, K_vals[0], K_vals[1])
    x1 = _acc1(x, rt_pk, rt_k0, rt_k1, rt_base)
    x2 = _acc2(x1, rt_pk, rt_k0, rt_k1, rt_base)
    enc, out = _mlp(x, x1, x2, W1, b1, W2, b2)
    return (enc, out)


# speedup vs baseline: 18.7883x; 18.7883x over previous
import jax
import jax.numpy as jnp
from jax import lax
from jax.experimental import pallas as pl
from jax.experimental.pallas import tpu as pltpu
from jax.experimental.pallas import tpu_sc as plsc

N = 10000
E = 320000
D = 128
NK = 2
H = 128
OUT = 64

NC = 2
NS = 16
NW = NC * NS
EPT = E // NW
NPAD = 10240
NRG = 64
RR = NPAD // NRG
NQ = 8
QR = NPAD // NQ
SB = 256
REPT = EPT + SB + 16
ROUNDS = NRG // NW

_mesh = plsc.VectorSubcoreMesh(core_axis_name="c", subcore_axis_name="s",
                               num_cores=NC, num_subcores=NS)
_sc_params = pltpu.CompilerParams(needs_layout_passes=False)


def _route_body(pki, k0i, k1i, rt_pk, rt_k0, rt_k1, rt_base,
                m_pk, m_k0, m_k1, m2_pk, m2_k0, m2_k1, bb):
    c = lax.axis_index("c")
    s = lax.axis_index("s")
    wid = s * NC + c
    ebase = wid * EPT
    zvec_i = jnp.zeros((16,), jnp.int32)

    pltpu.sync_copy(pki.at[pl.ds(pl.multiple_of(ebase, 8), EPT)],
                    m_pk.at[pl.ds(0, EPT)])
    pltpu.sync_copy(k0i.at[pl.ds(pl.multiple_of(ebase, 8), EPT)],
                    m_k0.at[pl.ds(0, EPT)])
    pltpu.sync_copy(k1i.at[pl.ds(pl.multiple_of(ebase, 8), EPT)],
                    m_k1.at[pl.ds(0, EPT)])

    def cnt1(v, cr):
        d = m_pk[pl.ds(v * 16, 16)] >> 16
        return tuple(
            cr[q] + plsc.all_reduce_population_count(
                (d >= q * QR) & (d < (q + 1) * QR))
            for q in range(NQ))

    qcnt = lax.fori_loop(0, EPT // 16, cnt1, (zvec_i,) * NQ)
    qbase = [zvec_i]
    for q in range(NQ - 1):
        qbase.append(qbase[q] + qcnt[q])

    def fill1(v, w):
        sl = pl.ds(v * 16, 16)
        pk = m_pk[sl]
        kv0 = m_k0[sl]
        kv1 = m_k1[sl]
        d = pk >> 16
        ws = list(w)
        for q in range(NQ):
            m = (d >= q * QR) & (d < (q + 1) * QR)
            mi = m.astype(jnp.int32)
            posv = ws[q] + lax.cumsum(mi) - mi
            plsc.store_scatter(m2_pk, [posv], pk, mask=m)
            plsc.store_scatter(m2_k0, [posv], kv0, mask=m)
            plsc.store_scatter(m2_k1, [posv], kv1, mask=m)
            ws[q] = ws[q] + plsc.all_reduce_population_count(m)
        return tuple(ws)

    lax.fori_loop(0, EPT // 16, fill1, tuple(qbase))

    iota16 = lax.iota(jnp.int32, 16)
    gbase_splat = []
    for q in range(NQ):
        qb = qbase[q][0]
        qe = qbase[q + 1][0] if q + 1 < NQ else jnp.int32(EPT)
        qa = pl.multiple_of(qb & jnp.int32(-8), 8)
        nv = (qe - qa + 15) // 16
        qbv = qbase[q]

        def cnt2(v, cr, qa=qa, qb=qb, qe=qe, q=q):
            voff = pl.multiple_of(qa + v * 16, 8)
            d = m2_pk[pl.ds(voff, 16)] >> 16
            posn = voff + iota16
            ok = (posn >= qb) & (posn < qe)
            return tuple(
                cr[k] + plsc.all_reduce_population_count(
                    ok & (d >= q * QR + k * RR) & (d < q * QR + (k + 1) * RR))
                for k in range(NQ))

        scnt = lax.fori_loop(0, nv, cnt2, (zvec_i,) * NQ)
        sbase = [qbv]
        for k in range(NQ - 1):
            sbase.append(sbase[k] + scnt[k])
        gbase_splat.extend(sbase)

        def fill2(v, w, qa=qa, qb=qb, qe=qe, q=q):
            voff = pl.multiple_of(qa + v * 16, 8)
            sl = pl.ds(voff, 16)
            pk = m2_pk[sl]
            kv0 = m2_k0[sl]
            kv1 = m2_k1[sl]
            d = pk >> 16
            posn = voff + iota16
            ok = (posn >= qb) & (posn < qe)
            ws = list(w)
            for k in range(NQ):
                m = ok & (d >= q * QR + k * RR) & (d < q * QR + (k + 1) * RR)
                mi = m.astype(jnp.int32)
                posv = ws[k] + lax.cumsum(mi) - mi
                plsc.store_scatter(m_pk, [posv], pk, mask=m)
                plsc.store_scatter(m_k0, [posv], kv0, mask=m)
                plsc.store_scatter(m_k1, [posv], kv1, mask=m)
                ws[k] = ws[k] + plsc.all_reduce_population_count(m)
            return tuple(ws)

        lax.fori_loop(0, nv, fill2, tuple(sbase))

    zvec_f = jnp.zeros((16,), jnp.float32)

    def ztail(i, _):
        t = pl.multiple_of(EPT + i * 16, 8)
        m_pk[pl.ds(t, 16)] = zvec_i
        m_k0[pl.ds(t, 16)] = zvec_f
        m_k1[pl.ds(t, 16)] = zvec_f
        return 0

    lax.fori_loop(0, (REPT - EPT) // 16, ztail, 0)

    lane0 = iota16 == 0
    for g in range(NRG):
        plsc.store_scatter(bb, [zvec_i + g], gbase_splat[g], mask=lane0)
    plsc.store_scatter(bb, [zvec_i + NRG], zvec_i + EPT, mask=lane0)
    pltpu.sync_copy(bb, rt_base.at[pl.ds(pl.multiple_of(wid * 80, 8), 80)])
    rbase = pl.multiple_of(wid * REPT, 8)
    pltpu.sync_copy(m_pk, rt_pk.at[pl.ds(rbase, REPT)])
    pltpu.sync_copy(m_k0, rt_k0.at[pl.ds(rbase, REPT)])
    pltpu.sync_copy(m_k1, rt_k1.at[pl.ds(rbase, REPT)])


_route = pl.kernel(
    _route_body,
    out_type=(jax.ShapeDtypeStruct((NW * REPT,), jnp.int32),
              jax.ShapeDtypeStruct((NW * REPT,), jnp.float32),
              jax.ShapeDtypeStruct((NW * REPT,), jnp.float32),
              jax.ShapeDtypeStruct((NW * 80,), jnp.int32)),
    mesh=_mesh,
    compiler_params=_sc_params,
    scratch_types=[
        pltpu.VMEM((REPT,), jnp.int32),
        pltpu.VMEM((REPT,), jnp.float32),
        pltpu.VMEM((REPT,), jnp.float32),
        pltpu.VMEM((EPT + 48,), jnp.int32),
        pltpu.VMEM((EPT + 48,), jnp.float32),
        pltpu.VMEM((EPT + 48,), jnp.float32),
        pltpu.VMEM((80,), jnp.int32),
    ],
)


def _make_acc(Din, C):
    Dout = NK * Din
    assert SB % C == 0 and C % 16 == 0

    def body(xin, rt_pk, rt_k0, rt_k1, rt_base, xout,
             tbv, s_pk, s_k0, s_k1, gidx, rows, acc):
        c = lax.axis_index("c")
        s = lax.axis_index("s")
        wid = s * NC + c
        zvec_f = jnp.zeros((16,), jnp.float32)
        iota16 = lax.iota(jnp.int32, 16)

        pltpu.sync_copy(rt_base, tbv)
        uof = jnp.zeros((16,), jnp.int32)

        for r in range(ROUNDS):
            g = wid + NW * r
            range_lo = g * RR

            def zacc(i, _):
                for qq in range(Dout // 16):
                    acc[i, pl.ds(qq * 16, 16)] = zvec_f
                return 0

            lax.fori_loop(0, RR, zacc, 0)

            def u_body(u, _):
                b0 = plsc.load_gather(tbv, [uof + u * 80 + g])[0]
                b1 = plsc.load_gather(tbv, [uof + u * 80 + g + 1])[0]
                astart = pl.multiple_of(b0 & jnp.int32(-8), 8)
                nblk = (b1 - astart + SB - 1) // SB

                def blk_body(bi, _2):
                    boff = pl.multiple_of(astart + bi * SB, 8)
                    fo = pl.multiple_of(u * REPT + boff, 8)
                    pltpu.sync_copy(rt_pk.at[pl.ds(fo, SB)], s_pk)
                    pltpu.sync_copy(rt_k0.at[pl.ds(fo, SB)], s_k0)
                    pltpu.sync_copy(rt_k1.at[pl.ds(fo, SB)], s_k1)

                    ne = jnp.minimum(b1, boff + SB) - boff
                    nch = (ne + C - 1) // C

                    def unpk(v, _3):
                        sl = pl.ds(v * 16, 16)
                        gidx[sl] = s_pk[sl] & jnp.int32(0xFFFF)
                        return 0

                    lax.fori_loop(0, nch * (C // 16), unpk, 0)

                    def ch_body(cc, _3):
                        coff = pl.multiple_of(cc * C, 8)
                        pltpu.sync_copy(xin.at[gidx.at[pl.ds(coff, C)]], rows)

                        def vec_body(v2, _4):
                            voff = pl.multiple_of(coff + v2 * 16, 8)
                            sl = pl.ds(voff, 16)
                            posn = boff + voff + iota16
                            ok = (posn >= b0) & (posn < b1)
                            kv0 = jnp.where(ok, s_k0[sl], 0.0)
                            kv1 = jnp.where(ok, s_k1[sl], 0.0)
                            dv = s_pk[sl] >> 16
                            liv = jnp.clip(dv - range_lo, 0, RR - 1)
                            for e in range(16):
                                er = v2 * 16 + e
                                li = liv[e]
                                k0s = kv0[e]
                                k1s = kv1[e]
                                for qd in range(Din // 16):
                                    rv = rows[er, pl.ds(qd * 16, 16)]
                                    plsc.addupdate(
                                        acc.at[li, pl.ds(qd * 16, 16)],
                                        rv * k0s)
                                    plsc.addupdate(
                                        acc.at[li, pl.ds(Din + qd * 16, 16)],
                                        rv * k1s)
                            return 0

                        lax.fori_loop(0, C // 16, vec_body, 0)
                        return 0

                    lax.fori_loop(0, nch, ch_body, 0)
                    return 0

                lax.fori_loop(0, nblk, blk_body, 0)
                return 0

            lax.fori_loop(0, NW, u_body, 0)
            pltpu.sync_copy(
                acc, xout.at[pl.ds(pl.multiple_of(range_lo, 8), RR)])

    return pl.kernel(
        body,
        out_type=jax.ShapeDtypeStruct((NPAD, Dout), jnp.float32),
        mesh=_mesh,
        compiler_params=_sc_params,
        scratch_types=[
            pltpu.VMEM((NW * 80,), jnp.int32),
            pltpu.VMEM((SB,), jnp.int32),
            pltpu.VMEM((SB,), jnp.float32),
            pltpu.VMEM((SB,), jnp.float32),
            pltpu.VMEM((SB,), jnp.int32),
            pltpu.VMEM((C, Din), jnp.float32),
            pltpu.VMEM((RR, Dout), jnp.float32),
        ],
    )


_acc1 = _make_acc(Din=D, C=64)
_acc2 = _make_acc(Din=NK * D, C=64)

BLK = 1000


def _mlp_body(x_ref, x1_ref, x2_ref, w1_ref, b1_ref, w2_ref, b2_ref,
              enc_ref, out_ref):
    x = x_ref[...]
    nrm = jnp.sqrt(jnp.sum(x * x, axis=1, keepdims=True))
    out0 = x / jnp.maximum(nrm, 1e-12)
    x1 = x1_ref[...]
    x2 = x2_ref[...]
    h = (jnp.dot(out0, w1_ref[0:D, :], preferred_element_type=jnp.float32)
         + jnp.dot(x1, w1_ref[D:D + NK * D, :], preferred_element_type=jnp.float32)
         + jnp.dot(x2, w1_ref[D + NK * D:, :], preferred_element_type=jnp.float32)
         + b1_ref[...])
    h = jnp.maximum(h, 0.0)
    enc_ref[...] = jnp.dot(h, w2_ref[...], preferred_element_type=jnp.float32) + b2_ref[...]
    out_ref[:, 0:D] = out0
    out_ref[:, D:D + NK * D] = x1
    out_ref[:, D + NK * D:] = x2


def _mlp(x, x1, x2, W1, b1, W2, b2):
    total = D + NK * D + NK * NK * D
    grid = N // BLK
    return pl.pallas_call(
        _mlp_body,
        grid=(grid,),
        in_specs=[
            pl.BlockSpec((BLK, D), lambda i: (i, 0)),
            pl.BlockSpec((BLK, NK * D), lambda i: (i, 0)),
            pl.BlockSpec((BLK, NK * NK * D), lambda i: (i, 0)),
            pl.BlockSpec((total, H), lambda i: (0, 0)),
            pl.BlockSpec((1, H), lambda i: (0, 0)),
            pl.BlockSpec((H, OUT), lambda i: (0, 0)),
            pl.BlockSpec((1, OUT), lambda i: (0, 0)),
        ],
        out_specs=[
            pl.BlockSpec((BLK, OUT), lambda i: (i, 0)),
            pl.BlockSpec((BLK, total), lambda i: (i, 0)),
        ],
        out_shape=[
            jax.ShapeDtypeStruct((N, OUT), jnp.float32),
            jax.ShapeDtypeStruct((N, total), jnp.float32),
        ],
    )(x, x1, x2, W1, b1.reshape(1, H), W2, b2.reshape(1, OUT))


@jax.jit
def kernel(x, edge_index, K_vals, W1, b1, W2, b2):
    src = edge_index[0]
    dst = edge_index[1]
    pk = src | (dst << 16)
    rt_pk, rt_k0, rt_k1, rt_base = _route(pk, K_vals[0], K_vals[1])
    x1 = _acc1(x, rt_pk, rt_k0, rt_k1, rt_base)
    x2 = _acc2(x1, rt_pk, rt_k0, rt_k1, rt_base)
    enc, out = _mlp(x, x1, x2, W1, b1, W2, b2)
    return (enc, out)


# speedup vs baseline: 19.7165x; 1.0494x over previous
import jax
import jax.numpy as jnp
from jax import lax
from jax.experimental import pallas as pl
from jax.experimental.pallas import tpu as pltpu
from jax.experimental.pallas import tpu_sc as plsc

N = 10000
E = 320000
D = 128
NK = 2
H = 128
OUT = 64

NC = 2
NS = 16
NW = NC * NS
EPT = E // NW
NPAD = 10240
NRG = 64
RR = NPAD // NRG
NQ = 8
QR = NPAD // NQ
SB = 256
REPT = EPT + SB + 16
ROUNDS = NRG // NW

_mesh = plsc.VectorSubcoreMesh(core_axis_name="c", subcore_axis_name="s",
                               num_cores=NC, num_subcores=NS)
_sc_params = pltpu.CompilerParams(needs_layout_passes=False)


def _route_body(pki, k0i, k1i, rt_pk, rt_k0, rt_k1, rt_base,
                m_pk, m_k0, m_k1, m2_pk, m2_k0, m2_k1, bb):
    c = lax.axis_index("c")
    s = lax.axis_index("s")
    wid = s * NC + c
    ebase = wid * EPT
    zvec_i = jnp.zeros((16,), jnp.int32)

    pltpu.sync_copy(pki.at[pl.ds(pl.multiple_of(ebase, 8), EPT)],
                    m_pk.at[pl.ds(0, EPT)])
    pltpu.sync_copy(k0i.at[pl.ds(pl.multiple_of(ebase, 8), EPT)],
                    m_k0.at[pl.ds(0, EPT)])
    pltpu.sync_copy(k1i.at[pl.ds(pl.multiple_of(ebase, 8), EPT)],
                    m_k1.at[pl.ds(0, EPT)])

    def cnt1(v, cr):
        d = m_pk[pl.ds(v * 16, 16)] >> 16
        return tuple(
            cr[q] + plsc.all_reduce_population_count(
                (d >= q * QR) & (d < (q + 1) * QR))
            for q in range(NQ))

    qcnt = lax.fori_loop(0, EPT // 16, cnt1, (zvec_i,) * NQ)
    qbase = [zvec_i]
    for q in range(NQ - 1):
        qbase.append(qbase[q] + qcnt[q])

    def fill1(v, w):
        sl = pl.ds(v * 16, 16)
        pk = m_pk[sl]
        kv0 = m_k0[sl]
        kv1 = m_k1[sl]
        d = pk >> 16
        ws = list(w)
        for q in range(NQ):
            m = (d >= q * QR) & (d < (q + 1) * QR)
            mi = m.astype(jnp.int32)
            posv = ws[q] + lax.cumsum(mi) - mi
            plsc.store_scatter(m2_pk, [posv], pk, mask=m)
            plsc.store_scatter(m2_k0, [posv], kv0, mask=m)
            plsc.store_scatter(m2_k1, [posv], kv1, mask=m)
            ws[q] = ws[q] + plsc.all_reduce_population_count(m)
        return tuple(ws)

    lax.fori_loop(0, EPT // 16, fill1, tuple(qbase))

    iota16 = lax.iota(jnp.int32, 16)
    gbase_splat = []
    for q in range(NQ):
        qb = qbase[q][0]
        qe = qbase[q + 1][0] if q + 1 < NQ else jnp.int32(EPT)
        qa = pl.multiple_of(qb & jnp.int32(-8), 8)
        nv = (qe - qa + 15) // 16
        qbv = qbase[q]

        def cnt2(v, cr, qa=qa, qb=qb, qe=qe, q=q):
            voff = pl.multiple_of(qa + v * 16, 8)
            d = m2_pk[pl.ds(voff, 16)] >> 16
            posn = voff + iota16
            ok = (posn >= qb) & (posn < qe)
            return tuple(
                cr[k] + plsc.all_reduce_population_count(
                    ok & (d >= q * QR + k * RR) & (d < q * QR + (k + 1) * RR))
                for k in range(NQ))

        scnt = lax.fori_loop(0, nv, cnt2, (zvec_i,) * NQ)
        sbase = [qbv]
        for k in range(NQ - 1):
            sbase.append(sbase[k] + scnt[k])
        gbase_splat.extend(sbase)

        def fill2(v, w, qa=qa, qb=qb, qe=qe, q=q):
            voff = pl.multiple_of(qa + v * 16, 8)
            sl = pl.ds(voff, 16)
            pk = m2_pk[sl]
            kv0 = m2_k0[sl]
            kv1 = m2_k1[sl]
            d = pk >> 16
            posn = voff + iota16
            ok = (posn >= qb) & (posn < qe)
            ws = list(w)
            for k in range(NQ):
                m = ok & (d >= q * QR + k * RR) & (d < q * QR + (k + 1) * RR)
                mi = m.astype(jnp.int32)
                posv = ws[k] + lax.cumsum(mi) - mi
                plsc.store_scatter(m_pk, [posv], pk, mask=m)
                plsc.store_scatter(m_k0, [posv], kv0, mask=m)
                plsc.store_scatter(m_k1, [posv], kv1, mask=m)
                ws[k] = ws[k] + plsc.all_reduce_population_count(m)
            return tuple(ws)

        lax.fori_loop(0, nv, fill2, tuple(sbase))

    zvec_f = jnp.zeros((16,), jnp.float32)

    def ztail(i, _):
        t = pl.multiple_of(EPT + i * 16, 8)
        m_pk[pl.ds(t, 16)] = zvec_i
        m_k0[pl.ds(t, 16)] = zvec_f
        m_k1[pl.ds(t, 16)] = zvec_f
        return 0

    lax.fori_loop(0, (REPT - EPT) // 16, ztail, 0)

    lane0 = iota16 == 0
    for g in range(NRG):
        plsc.store_scatter(bb, [zvec_i + g], gbase_splat[g], mask=lane0)
    plsc.store_scatter(bb, [zvec_i + NRG], zvec_i + EPT, mask=lane0)
    pltpu.sync_copy(bb, rt_base.at[pl.ds(pl.multiple_of(wid * 80, 8), 80)])
    rbase = pl.multiple_of(wid * REPT, 8)
    pltpu.sync_copy(m_pk, rt_pk.at[pl.ds(rbase, REPT)])
    pltpu.sync_copy(m_k0, rt_k0.at[pl.ds(rbase, REPT)])
    pltpu.sync_copy(m_k1, rt_k1.at[pl.ds(rbase, REPT)])


_route = pl.kernel(
    _route_body,
    out_type=(jax.ShapeDtypeStruct((NW * REPT,), jnp.int32),
              jax.ShapeDtypeStruct((NW * REPT,), jnp.float32),
              jax.ShapeDtypeStruct((NW * REPT,), jnp.float32),
              jax.ShapeDtypeStruct((NW * 80,), jnp.int32)),
    mesh=_mesh,
    compiler_params=_sc_params,
    scratch_types=[
        pltpu.VMEM((REPT,), jnp.int32),
        pltpu.VMEM((REPT,), jnp.float32),
        pltpu.VMEM((REPT,), jnp.float32),
        pltpu.VMEM((EPT + 48,), jnp.int32),
        pltpu.VMEM((EPT + 48,), jnp.float32),
        pltpu.VMEM((EPT + 48,), jnp.float32),
        pltpu.VMEM((80,), jnp.int32),
    ],
)


def _make_acc(Din, C):
    Dout = NK * Din
    assert SB % C == 0 and C % 16 == 0

    def body(xin, rt_pk, rt_k0, rt_k1, rt_base, xout,
             tbv, s_pk, s_k0, s_k1, gidx, rows, acc, sem0, sem1, sem2):
        c = lax.axis_index("c")
        s = lax.axis_index("s")
        wid = s * NC + c
        zvec_f = jnp.zeros((16,), jnp.float32)
        iota16 = lax.iota(jnp.int32, 16)

        pltpu.sync_copy(rt_base, tbv)
        uof = jnp.zeros((16,), jnp.int32)

        for r in range(ROUNDS):
            g = wid + NW * r
            range_lo = g * RR

            def zacc(i, _):
                for qq in range(Dout // 16):
                    acc[i, pl.ds(qq * 16, 16)] = zvec_f
                return 0

            lax.fori_loop(0, RR, zacc, 0)

            def u_body(u, _):
                b0 = plsc.load_gather(tbv, [uof + u * 80 + g])[0]
                b1 = plsc.load_gather(tbv, [uof + u * 80 + g + 1])[0]
                astart = pl.multiple_of(b0 & jnp.int32(-8), 8)
                nblk = (b1 - astart + SB - 1) // SB

                def blk_body(bi, _2):
                    boff = pl.multiple_of(astart + bi * SB, 8)
                    fo = pl.multiple_of(u * REPT + boff, 8)
                    d0 = pltpu.async_copy(rt_pk.at[pl.ds(fo, SB)], s_pk, sem0)
                    d1 = pltpu.async_copy(rt_k0.at[pl.ds(fo, SB)], s_k0, sem1)
                    d2 = pltpu.async_copy(rt_k1.at[pl.ds(fo, SB)], s_k1, sem2)
                    d0.wait()
                    d1.wait()
                    d2.wait()

                    ne = jnp.minimum(b1, boff + SB) - boff
                    nch = (ne + C - 1) // C

                    def unpk(v, _3):
                        sl = pl.ds(v * 16, 16)
                        gidx[sl] = s_pk[sl] & jnp.int32(0xFFFF)
                        return 0

                    lax.fori_loop(0, nch * (C // 16), unpk, 0)

                    def ch_body(cc, _3):
                        coff = pl.multiple_of(cc * C, 8)
                        pltpu.sync_copy(xin.at[gidx.at[pl.ds(coff, C)]], rows)

                        def vec_body(v2, _4):
                            voff = pl.multiple_of(coff + v2 * 16, 8)
                            sl = pl.ds(voff, 16)
                            posn = boff + voff + iota16
                            ok = (posn >= b0) & (posn < b1)
                            kv0 = jnp.where(ok, s_k0[sl], 0.0)
                            kv1 = jnp.where(ok, s_k1[sl], 0.0)
                            dv = s_pk[sl] >> 16
                            liv = jnp.clip(dv - range_lo, 0, RR - 1)
                            for e in range(16):
                                er = v2 * 16 + e
                                li = liv[e]
                                k0s = kv0[e]
                                k1s = kv1[e]
                                for qd in range(Din // 16):
                                    rv = rows[er, pl.ds(qd * 16, 16)]
                                    plsc.addupdate(
                                        acc.at[li, pl.ds(qd * 16, 16)],
                                        rv * k0s)
                                    plsc.addupdate(
                                        acc.at[li, pl.ds(Din + qd * 16, 16)],
                                        rv * k1s)
                            return 0

                        lax.fori_loop(0, C // 16, vec_body, 0)
                        return 0

                    lax.fori_loop(0, nch, ch_body, 0)
                    return 0

                lax.fori_loop(0, nblk, blk_body, 0)
                return 0

            lax.fori_loop(0, NW, u_body, 0)
            pltpu.sync_copy(
                acc, xout.at[pl.ds(pl.multiple_of(range_lo, 8), RR)])

    return pl.kernel(
        body,
        out_type=jax.ShapeDtypeStruct((NPAD, Dout), jnp.float32),
        mesh=_mesh,
        compiler_params=_sc_params,
        scratch_types=[
            pltpu.VMEM((NW * 80,), jnp.int32),
            pltpu.VMEM((SB,), jnp.int32),
            pltpu.VMEM((SB,), jnp.float32),
            pltpu.VMEM((SB,), jnp.float32),
            pltpu.VMEM((SB,), jnp.int32),
            pltpu.VMEM((C, Din), jnp.float32),
            pltpu.VMEM((RR, Dout), jnp.float32),
            pltpu.SemaphoreType.DMA,
            pltpu.SemaphoreType.DMA,
            pltpu.SemaphoreType.DMA,
        ],
    )


_acc1 = _make_acc(Din=D, C=64)
_acc2 = _make_acc(Din=NK * D, C=64)

BLK = 1000


def _mlp_body(x_ref, x1_ref, x2_ref, w1_ref, b1_ref, w2_ref, b2_ref,
              enc_ref, out_ref):
    x = x_ref[...]
    nrm = jnp.sqrt(jnp.sum(x * x, axis=1, keepdims=True))
    out0 = x / jnp.maximum(nrm, 1e-12)
    x1 = x1_ref[...]
    x2 = x2_ref[...]
    h = (jnp.dot(out0, w1_ref[0:D, :], preferred_element_type=jnp.float32)
         + jnp.dot(x1, w1_ref[D:D + NK * D, :], preferred_element_type=jnp.float32)
         + jnp.dot(x2, w1_ref[D + NK * D:, :], preferred_element_type=jnp.float32)
         + b1_ref[...])
    h = jnp.maximum(h, 0.0)
    enc_ref[...] = jnp.dot(h, w2_ref[...], preferred_element_type=jnp.float32) + b2_ref[...]
    out_ref[:, 0:D] = out0
    out_ref[:, D:D + NK * D] = x1
    out_ref[:, D + NK * D:] = x2


def _mlp(x, x1, x2, W1, b1, W2, b2):
    total = D + NK * D + NK * NK * D
    grid = N // BLK
    return pl.pallas_call(
        _mlp_body,
        grid=(grid,),
        in_specs=[
            pl.BlockSpec((BLK, D), lambda i: (i, 0)),
            pl.BlockSpec((BLK, NK * D), lambda i: (i, 0)),
            pl.BlockSpec((BLK, NK * NK * D), lambda i: (i, 0)),
            pl.BlockSpec((total, H), lambda i: (0, 0)),
            pl.BlockSpec((1, H), lambda i: (0, 0)),
            pl.BlockSpec((H, OUT), lambda i: (0, 0)),
            pl.BlockSpec((1, OUT), lambda i: (0, 0)),
        ],
        out_specs=[
            pl.BlockSpec((BLK, OUT), lambda i: (i, 0)),
            pl.BlockSpec((BLK, total), lambda i: (i, 0)),
        ],
        out_shape=[
            jax.ShapeDtypeStruct((N, OUT), jnp.float32),
            jax.ShapeDtypeStruct((N, total), jnp.float32),
        ],
    )(x, x1, x2, W1, b1.reshape(1, H), W2, b2.reshape(1, OUT))


@jax.jit
def kernel(x, edge_index, K_vals, W1, b1, W2, b2):
    src = edge_index[0]
    dst = edge_index[1]
    pk = src | (dst << 16)
    rt_pk, rt_k0, rt_k1, rt_base = _route(pk, K_vals[0], K_vals[1])
    x1 = _acc1(x, rt_pk, rt_k0, rt_k1, rt_base)
    x2 = _acc2(x1, rt_pk, rt_k0, rt_k1, rt_base)
    enc, out = _mlp(x, x1, x2, W1, b1, W2, b2)
    return (enc, out)
